# baseline (speedup 1.0000x reference)
import functools
import numpy as np

import jax
import jax.numpy as jnp
from jax import lax
from jax.experimental import pallas as pl
from jax.experimental.pallas import tpu as pltpu
from jax.experimental.pallas import tpu_sc as plsc

_N = 10000
_NP = 12288
_Q = 100000
_QP = 102400
_D = 256
_B = 4
_E = 160000
_CAP = 32

_NC = 2
_NS = 16
_NW = _NC * _NS

_CPT = _NP // _NW
_WQ = 2048
_WQA = 1024
_WE = 3200

_KE = 128
_NWIN = 80
_EPT = _KE * _NWIN
_E2 = _EPT * _NS
_ACC_ROWS = 10240
_ROWS_PT = _ACC_ROWS // _NS

_LANE = lambda: lax.broadcasted_iota(jnp.int32, (16,), 0)

_GDN = lax.GatherDimensionNumbers(offset_dims=(), collapsed_slice_dims=(0,),
                                  start_index_map=(0,))


def _take16(x, idx):
    return lax.gather(x, idx[:, None], _GDN, (1,),
                      mode=lax.GatherScatterMode.PROMISE_IN_BOUNDS)

_MESH = plsc.VectorSubcoreMesh(core_axis_name="c", subcore_axis_name="s")
_SC_PARAMS = pltpu.CompilerParams(needs_layout_passes=False)

_NEGINF = float("-inf")


def _wid():
    return lax.axis_index("s") * _NC + lax.axis_index("c")


def _fold_rows(hist_ref, out_ref, n_rows, extra):
    lane = _LANE()
    m0 = lane == 0
    perms = [lane ^ s for s in (1, 2, 4, 8)]

    def fold_body(r, _):
        x = hist_ref[pl.ds(r * 16, 16)]
        for p in perms:
            x = x + _take16(x, p)
        plsc.store_scatter(out_ref, [jnp.broadcast_to(r, (16,))], x + extra,
                           mask=m0)
        return _

    lax.fori_loop(0, n_rows, fold_body, None)


def _stats_body(dsts, ws, deg_out, hist, dwin, wwin, cl):
    wid = _wid()
    base = wid * _CPT
    lane = _LANE()
    zero16 = jnp.zeros((16,), jnp.float32)

    def zero_hist(r, _):
        hist[pl.ds(r * 16, 16)] = zero16
        return _

    lax.fori_loop(0, _CPT, zero_hist, None)

    def edge_win(k, _):
        e0 = k * _WE
        pltpu.sync_copy(dsts.at[pl.ds(e0, _WE)], dwin)
        pltpu.sync_copy(ws.at[pl.ds(e0, _WE)], wwin)

        def grp(j, _):
            dv = dwin[pl.ds(j * 16, 16)]
            wv = wwin[pl.ds(j * 16, 16)]
            rel = dv - base
            m = (rel >= 0) & (rel < _CPT)
            relc = jnp.where(m, rel, 0)
            plsc.addupdate_scatter(hist, [relc * 16 + lane], wv, mask=m)
            return _

        lax.fori_loop(0, _WE // 16, grp, None)
        return _

    lax.fori_loop(0, _E // _WE, edge_win, None)
    _fold_rows(hist, cl, _CPT, 1.0)
    pltpu.sync_copy(cl, deg_out.at[pl.ds(base, _CPT)])


_stats_call = pl.kernel(
    _stats_body,
    out_type=jax.ShapeDtypeStruct((_NP,), jnp.float32),
    mesh=_MESH,
    compiler_params=_SC_PARAMS,
    scratch_types=(
        pltpu.VMEM((_CPT * 16,), jnp.float32),
        pltpu.VMEM((_WE,), jnp.int32),
        pltpu.VMEM((_WE,), jnp.float32),
        pltpu.VMEM((_CPT,), jnp.float32),
    ),
)


def _stage_a_body(allocs, embT, segT, counts_out, acc0, acc1, acc2, acc3,
                  hist, cl, aw, ew, sems):
    wid = _wid()
    f0 = wid * 8
    base = wid * _CPT
    neg16 = jnp.full((16,), _NEGINF, jnp.float32)
    zero16 = jnp.zeros((16,), jnp.float32)
    one16 = jnp.full((16,), 1.0, jnp.float32)
    lane = _LANE()
    qsel1 = lax.shift_right_logical(lane, 1)
    fl2 = lane & 1
    accs = (acc0, acc1, acc2, acc3)
    nwin = _QP // _WQA
    half = nwin // 2

    def start(b, w, k):
        q0 = w * _WQA
        pltpu.async_copy(allocs.at[b].at[pl.ds(q0, _WQA)], aw[k], sems[k])
        pltpu.async_copy(embT.at[pl.ds(f0, 8), pl.ds(q0, _WQA)], ew[k],
                         sems[k])

    def wait(b, w, k):
        q0 = w * _WQA
        pltpu.make_async_copy(allocs.at[b].at[pl.ds(q0, _WQA)], aw[k],
                              sems[k]).wait()
        pltpu.make_async_copy(embT.at[pl.ds(f0, 8), pl.ds(q0, _WQA)], ew[k],
                              sems[k]).wait()

    def process(k):
        awin = aw[k]
        embw = ew[k]

        def oct_grp(j, _):
            av = awin[pl.ds(j * 16, 16)]
            rel = av - base
            m = (rel >= 0) & (rel < _CPT)
            relc = jnp.where(m, rel, 0)
            plsc.addupdate_scatter(hist, [relc * 16 + lane], one16, mask=m)
            for u in range(2):
                i = j * 2 + u
                qidx = qsel1 + 8 * i
                a8 = plsc.load_gather(awin, [qidx])
                vs = [plsc.load_gather(embw, [fl2 + 2 * c, qidx])
                      for c in range(4)]
                eqs = [a8 == _take16(a8, lane ^ (2 * s))
                       for s in range(1, 8)]
                dup = eqs[0]
                for e in eqs[1:]:
                    dup = dup | e
                has_dup = jnp.any(dup)

                def combine(vals):
                    for s in range(1, 8):
                        vals = [jnp.where(eqs[s - 1],
                                          jnp.maximum(
                                              v, _take16(v, lane ^ (2 * s))),
                                          v)
                                for v in vals]
                    return tuple(vals)

                vs = lax.cond(has_dup, lambda: combine(vs), lambda: tuple(vs))
                for c, (ac, v) in enumerate(zip(accs, vs)):
                    o = plsc.load_gather(ac, [fl2, a8])
                    plsc.store_scatter(ac, [fl2, a8], jnp.maximum(o, v))
            return _

        lax.fori_loop(0, _WQA // 16, oct_grp, None)

    for b in range(_B):
        def acc_init(i, _):
            for ac in accs:
                for f in range(2):
                    ac[f, pl.ds(i * 16, 16)] = neg16
            return _

        lax.fori_loop(0, _NP // 16, acc_init, None)

        def zero_hist(r, _):
            hist[pl.ds(r * 16, 16)] = zero16
            return _

        lax.fori_loop(0, _CPT, zero_hist, None)

        start(b, 0, 0)

        def wpair(wp, _):
            w0 = 2 * wp
            start(b, w0 + 1, 1)
            wait(b, w0, 0)
            process(0)

            @pl.when(wp + 1 < half)
            def _():
                start(b, w0 + 2, 0)

            wait(b, w0 + 1, 1)
            process(1)
            return _

        lax.fori_loop(0, half, wpair, None)
        for c, ac in enumerate(accs):
            pltpu.sync_copy(ac, segT.at[b].at[pl.ds(f0 + 2 * c, 2), :])
        _fold_rows(hist, cl, _CPT, 0.0)
        pltpu.sync_copy(cl, counts_out.at[b].at[pl.ds(base, _CPT)])


_stage_a_call = pl.kernel(
    _stage_a_body,
    out_type=(
        jax.ShapeDtypeStruct((_B, _D, _NP), jnp.float32),
        jax.ShapeDtypeStruct((_B, _NP), jnp.float32),
    ),
    mesh=_MESH,
    compiler_params=_SC_PARAMS,
    scratch_types=(
        pltpu.VMEM((2, _NP), jnp.float32),
        pltpu.VMEM((2, _NP), jnp.float32),
        pltpu.VMEM((2, _NP), jnp.float32),
        pltpu.VMEM((2, _NP), jnp.float32),
        pltpu.VMEM((_CPT * 16,), jnp.float32),
        pltpu.VMEM((_CPT,), jnp.float32),
        (pltpu.VMEM((_WQA,), jnp.int32), pltpu.VMEM((_WQA,), jnp.int32)),
        (pltpu.VMEM((8, _WQA), jnp.float32),
         pltpu.VMEM((8, _WQA), jnp.float32)),
        (pltpu.SemaphoreType.DMA, pltpu.SemaphoreType.DMA),
    ),
)


def _prop_body(*refs):
    g_refs = refs[0:8]
    srcs = refs[8]
    dsts = refs[9]
    ws = refs[10]
    s_refs = refs[11:19]
    (acc, rows, srcwin, dstwin, wwin, zbuf,
     isems, gsems, ssems, zsem) = refs[19:29]

    cid = lax.axis_index("c")
    sid = lax.axis_index("s")
    e0 = sid * _EPT
    zero16 = jnp.zeros((16,), jnp.float32)

    pltpu.sync_copy(ws.at[pl.ds(e0, _EPT)], wwin)

    def zero_zbuf(i, _):
        for f in range(8):
            zbuf[i, pl.ds(f * 16, 16)] = zero16
        return _

    lax.fori_loop(0, 16, zero_zbuf, None)

    def istage(w, i):
        we0 = e0 + w * _KE
        pltpu.async_copy(srcs.at[pl.ds(we0, _KE)], srcwin[i], isems[i])
        pltpu.async_copy(dsts.at[pl.ds(we0, _KE)], dstwin[i], isems[i])

    def iwait(w, i):
        we0 = e0 + w * _KE
        pltpu.make_async_copy(srcs.at[pl.ds(we0, _KE)], srcwin[i],
                              isems[i]).wait()
        pltpu.make_async_copy(dsts.at[pl.ds(we0, _KE)], dstwin[i],
                              isems[i]).wait()

    def gstart(gr, i, k):
        pltpu.async_copy(gr.at[srcwin[i]], rows[k], gsems[k])

    def gwait(gr, i, k):
        pltpu.make_async_copy(gr.at[srcwin[i]], rows[k], gsems[k]).wait()

    def scale(w, k):
        def grp(j, _):
            w16 = wwin[pl.ds(w * _KE + j * 16, 16)]
            for u in range(16):
                e = j * 16 + u
                wb = _take16(w16, jnp.full((16,), u, jnp.int32))
                for f in range(8):
                    rows[k][e, pl.ds(f * 16, 16)] = (
                        rows[k][e, pl.ds(f * 16, 16)] * wb)
            return _

        lax.fori_loop(0, _KE // 16, grp, None)

    def scat(i, k):
        pltpu.async_copy(rows[k], acc.at[dstwin[i]], ssems[k], add=True)

    def swait(i, k):
        pltpu.make_async_copy(rows[k], acc.at[dstwin[i]], ssems[k]).wait()

    for chunk in range(8):
        @pl.when(cid == chunk // 4)
        def _():
            gr = g_refs[chunk]
            sr = s_refs[chunk]

            def zero_acc(i, _):
                pltpu.async_copy(
                    zbuf, acc.at[pl.ds(sid * _ROWS_PT + i * 16, 16), :], zsem)
                return _

            lax.fori_loop(0, _ROWS_PT // 16, zero_acc, None)

            def zero_drain(i, _):
                pltpu.make_async_copy(
                    zbuf, acc.at[pl.ds(sid * _ROWS_PT + i * 16, 16), :],
                    zsem).wait()
                return _

            lax.fori_loop(0, _ROWS_PT // 16, zero_drain, None)
            plsc.subcore_barrier()

            istage(0, 0)
            iwait(0, 0)
            gstart(gr, 0, 0)
            istage(1, 1)

            def w4(wt, _):
                w0 = 4 * wt
                for u in range(4):
                    w = w0 + u
                    k = u % 2
                    iu = u
                    iun = (u + 1) % 4
                    iup = (u + 3) % 4

                    @pl.when(w + 1 < _NWIN)
                    def _():
                        iwait(w + 1, iun)

                    @pl.when(w + 2 < _NWIN)
                    def _():
                        istage(w + 2, (u + 2) % 4)

                    gwait(gr, iu, k)

                    @pl.when(w >= 1)
                    def _():
                        swait(iup, 1 - k)

                    @pl.when(w + 1 < _NWIN)
                    def _():
                        gstart(gr, iun, 1 - k)

                    scale(w, k)
                    scat(iu, k)
                return _

            lax.fori_loop(0, _NWIN // 4, w4, None)
            swait(3, 1)
            plsc.subcore_barrier()
            pltpu.sync_copy(acc.at[pl.ds(sid * _ROWS_PT, _ROWS_PT), :],
                            sr.at[pl.ds(sid * _ROWS_PT, _ROWS_PT), :])
            plsc.subcore_barrier()


_prop_call = pl.kernel(
    _prop_body,
    out_type=tuple(jax.ShapeDtypeStruct((_NP, 128), jnp.float32)
                   for _ in range(8)),
    mesh=_MESH,
    compiler_params=_SC_PARAMS,
    scratch_types=(
        pltpu.VMEM_SHARED((_ACC_ROWS, 128), jnp.float32),
        (pltpu.VMEM((_KE, 128), jnp.float32),
         pltpu.VMEM((_KE, 128), jnp.float32)),
        tuple(pltpu.VMEM((_KE,), jnp.int32) for _ in range(4)),
        tuple(pltpu.VMEM((_KE,), jnp.int32) for _ in range(4)),
        pltpu.VMEM((_EPT,), jnp.float32),
        pltpu.VMEM((16, 128), jnp.float32),
        tuple(pltpu.SemaphoreType.DMA for _ in range(4)),
        (pltpu.SemaphoreType.DMA, pltpu.SemaphoreType.DMA),
        (pltpu.SemaphoreType.DMA, pltpu.SemaphoreType.DMA),
        pltpu.SemaphoreType.DMA,
    ),
)


_BM = 512
_GRID = _NP // _BM


def _mm1_body(seg_ref, cnt_ref, dummy_ref, dinv_ref, w_ref, *out_refs):
    w = w_ref[...]
    dummy = dummy_ref[...]
    dinv = dinv_ref[...]
    for b in range(_B):
        x = seg_ref[b]
        cnt = cnt_ref[pl.ds(b, 1), :]
        xm = jnp.where(cnt < float(_CAP), jnp.maximum(x, dummy), x)
        h = lax.dot_general(xm, w, (((0,), (0,)), ((), ())),
                            preferred_element_type=jnp.float32)
        g = h * dinv
        out_refs[2 * b][...] = g[:, :128]
        out_refs[2 * b + 1][...] = g[:, 128:]


def _mm1(segT, counts, dummy2d, dinv2d, W1):
    return pl.pallas_call(
        _mm1_body,
        grid=(_GRID,),
        in_specs=[
            pl.BlockSpec((_B, _D, _BM), lambda m: (0, 0, m)),
            pl.BlockSpec((_B, _BM), lambda m: (0, m)),
            pl.BlockSpec((_D, 1), lambda m: (0, 0)),
            pl.BlockSpec((_BM, 1), lambda m: (m, 0)),
            pl.BlockSpec((_D, _D), lambda m: (0, 0)),
        ],
        out_specs=[pl.BlockSpec((_BM, 128), lambda m: (m, 0))
                   for _ in range(8)],
        out_shape=[jax.ShapeDtypeStruct((_NP, 128), jnp.float32)
                   for _ in range(8)],
    )(segT, counts, dummy2d, dinv2d, W1)


def _mm2_body(dinv_ref, bias_ref, w_ref, *refs):
    s_refs = refs[0:8]
    g_refs = refs[8:16]
    out_refs = refs[16:24]
    w = w_ref[...]
    bias = bias_ref[...]
    dinv = dinv_ref[...]
    for b in range(_B):
        s = jnp.concatenate([s_refs[2 * b][...], s_refs[2 * b + 1][...]],
                            axis=1)
        g = jnp.concatenate([g_refs[2 * b][...], g_refs[2 * b + 1][...]],
                            axis=1)
        x = jnp.maximum(dinv * (s + g) + bias, 0.0)
        h = lax.dot_general(x, w, (((1,), (0,)), ((), ())),
                            preferred_element_type=jnp.float32)
        g2 = h * dinv
        out_refs[2 * b][...] = g2[:, :128]
        out_refs[2 * b + 1][...] = g2[:, 128:]


def _mm2(s_list, g_list, dinv2d, bias2d, W2):
    blk = pl.BlockSpec((_BM, 128), lambda m: (m, 0))
    return pl.pallas_call(
        _mm2_body,
        grid=(_GRID,),
        in_specs=[
            pl.BlockSpec((_BM, 1), lambda m: (m, 0)),
            pl.BlockSpec((1, _D), lambda m: (0, 0)),
            pl.BlockSpec((_D, _D), lambda m: (0, 0)),
        ] + [blk] * 16,
        out_specs=[pl.BlockSpec((_BM, 128), lambda m: (m, 0))
                   for _ in range(8)],
        out_shape=[jax.ShapeDtypeStruct((_NP, 128), jnp.float32)
                   for _ in range(8)],
    )(dinv2d, bias2d, W2, *s_list, *g_list)


def _fin_body(dinv_ref, bias_ref, *refs):
    s_refs = refs[0:8]
    g_refs = refs[8:16]
    out_ref = refs[16]
    bias = bias_ref[...]
    dinv = dinv_ref[...]
    for b in range(_B):
        s = jnp.concatenate([s_refs[2 * b][...], s_refs[2 * b + 1][...]],
                            axis=1)
        g = jnp.concatenate([g_refs[2 * b][...], g_refs[2 * b + 1][...]],
                            axis=1)
        out_ref[b, :, :] = jnp.maximum(dinv * (s + g) + bias, 0.0)


def _fin(s_list, g_list, dinv2d, bias2d):
    blk = pl.BlockSpec((_BM, 128), lambda m: (m, 0))
    return pl.pallas_call(
        _fin_body,
        grid=(_GRID,),
        in_specs=[
            pl.BlockSpec((_BM, 1), lambda m: (m, 0)),
            pl.BlockSpec((1, _D), lambda m: (0, 0)),
        ] + [blk] * 16,
        out_specs=pl.BlockSpec((_B, _BM, _D), lambda m: (0, m, 0)),
        out_shape=jax.ShapeDtypeStruct((_B, _NP, _D), jnp.float32),
    )(dinv2d, bias2d, *s_list, *g_list)


def kernel(core_allocs, qubit_embs, dummy_qubit_emb, edge_index, edge_weight,
           W1, b1, W2, b2):
    src = edge_index[0].astype(jnp.int32)
    dst = edge_index[1].astype(jnp.int32)
    qpad = _QP - _Q
    alloc_pad = _N + (jnp.arange(qpad, dtype=jnp.int32) % (_NP - _N))
    allocs = jnp.concatenate(
        [core_allocs.astype(jnp.int32),
         jnp.broadcast_to(alloc_pad, (_B, qpad))], axis=1)
    embT = jnp.concatenate(
        [qubit_embs, jnp.zeros((qpad, _D), jnp.float32)], axis=0).T

    deg = _stats_call(dst, edge_weight)
    dinv2d = lax.rsqrt(deg)[:, None]

    segT, counts = _stage_a_call(allocs, embT)

    g_list = _mm1(segT, counts, dummy_qubit_emb[:, None], dinv2d, W1)

    npad = _E2 - _E
    pad_idx = (jnp.arange(npad, dtype=jnp.int32) * 16) % _N
    src2 = jnp.concatenate([src, pad_idx])
    dst2 = jnp.concatenate([dst, pad_idx])
    w2 = jnp.concatenate([edge_weight, jnp.zeros((npad,), jnp.float32)])

    s_list = _prop_call(*g_list, src2, dst2, w2)
    g2_list = _mm2(s_list, g_list, dinv2d, b1[None, :], W2)
    s2_list = _prop_call(*g2_list, src2, dst2, w2)
    return _fin(s2_list, g2_list, dinv2d, b2[None, :])[:, :_N, :]


# speedup vs baseline: 1.2221x; 1.2221x over previous
import functools
import numpy as np

import jax
import jax.numpy as jnp
from jax import lax
from jax.experimental import pallas as pl
from jax.experimental.pallas import tpu as pltpu
from jax.experimental.pallas import tpu_sc as plsc

_N = 10000
_NP = 12288
_Q = 100000
_QP = 102400
_D = 256
_B = 4
_E = 160000
_CAP = 32

_NC = 2
_NS = 16
_NW = _NC * _NS

_CPT = _NP // _NW
_WQ = 2048
_WQA = 1024
_WE = 3200

_KE = 128
_NWIN = 80
_EPT = _KE * _NWIN
_E2 = _EPT * _NS
_ACC_ROWS = 10240
_ROWS_PT = _ACC_ROWS // _NS

_LANE = lambda: lax.broadcasted_iota(jnp.int32, (16,), 0)

_GDN = lax.GatherDimensionNumbers(offset_dims=(), collapsed_slice_dims=(0,),
                                  start_index_map=(0,))


def _take16(x, idx):
    return lax.gather(x, idx[:, None], _GDN, (1,),
                      mode=lax.GatherScatterMode.PROMISE_IN_BOUNDS)

_MESH = plsc.VectorSubcoreMesh(core_axis_name="c", subcore_axis_name="s")
_SC_PARAMS = pltpu.CompilerParams(needs_layout_passes=False)

_NEGINF = float("-inf")


def _wid():
    return lax.axis_index("s") * _NC + lax.axis_index("c")


def _fold_rows(hist_ref, out_ref, n_rows, extra):
    lane = _LANE()
    m0 = lane == 0
    perms = [lane ^ s for s in (1, 2, 4, 8)]

    def fold_body(r, _):
        x = hist_ref[pl.ds(r * 16, 16)]
        for p in perms:
            x = x + _take16(x, p)
        plsc.store_scatter(out_ref, [jnp.broadcast_to(r, (16,))], x + extra,
                           mask=m0)
        return _

    lax.fori_loop(0, n_rows, fold_body, None)


def _stats_body(dsts, ws, deg_out, hist, dwin, wwin, cl):
    wid = _wid()
    base = wid * _CPT
    lane = _LANE()
    zero16 = jnp.zeros((16,), jnp.float32)

    def zero_hist(r, _):
        hist[pl.ds(r * 16, 16)] = zero16
        return _

    lax.fori_loop(0, _CPT, zero_hist, None)

    def edge_win(k, _):
        e0 = k * _WE
        pltpu.sync_copy(dsts.at[pl.ds(e0, _WE)], dwin)
        pltpu.sync_copy(ws.at[pl.ds(e0, _WE)], wwin)

        def grp(j, _):
            dv = dwin[pl.ds(j * 16, 16)]
            wv = wwin[pl.ds(j * 16, 16)]
            rel = dv - base
            m = (rel >= 0) & (rel < _CPT)
            relc = jnp.where(m, rel, 0)
            plsc.addupdate_scatter(hist, [relc * 16 + lane], wv, mask=m)
            return _

        lax.fori_loop(0, _WE // 16, grp, None)
        return _

    lax.fori_loop(0, _E // _WE, edge_win, None)
    _fold_rows(hist, cl, _CPT, 1.0)
    pltpu.sync_copy(cl, deg_out.at[pl.ds(base, _CPT)])


_stats_call = pl.kernel(
    _stats_body,
    out_type=jax.ShapeDtypeStruct((_NP,), jnp.float32),
    mesh=_MESH,
    compiler_params=_SC_PARAMS,
    scratch_types=(
        pltpu.VMEM((_CPT * 16,), jnp.float32),
        pltpu.VMEM((_WE,), jnp.int32),
        pltpu.VMEM((_WE,), jnp.float32),
        pltpu.VMEM((_CPT,), jnp.float32),
    ),
)


def _stage_a_body(allocs, embT, segT, counts_out, acc0, acc1, acc2, acc3,
                  hist, cl, aw, ew, sems):
    wid = _wid()
    f0 = wid * 8
    base = wid * _CPT
    neg16 = jnp.full((16,), _NEGINF, jnp.float32)
    zero16 = jnp.zeros((16,), jnp.float32)
    one16 = jnp.full((16,), 1.0, jnp.float32)
    lane = _LANE()
    qsel1 = lax.shift_right_logical(lane, 1)
    fl2 = lane & 1
    accs = (acc0, acc1, acc2, acc3)
    nwin = _QP // _WQA
    half = nwin // 2

    def start(b, w, k):
        q0 = w * _WQA
        pltpu.async_copy(allocs.at[b].at[pl.ds(q0, _WQA)], aw[k], sems[k])
        pltpu.async_copy(embT.at[pl.ds(f0, 8), pl.ds(q0, _WQA)], ew[k],
                         sems[k])

    def wait(b, w, k):
        q0 = w * _WQA
        pltpu.make_async_copy(allocs.at[b].at[pl.ds(q0, _WQA)], aw[k],
                              sems[k]).wait()
        pltpu.make_async_copy(embT.at[pl.ds(f0, 8), pl.ds(q0, _WQA)], ew[k],
                              sems[k]).wait()

    def process(k):
        awin = aw[k]
        embw = ew[k]

        def oct_grp(j, _):
            av = awin[pl.ds(j * 16, 16)]
            rel = av - base
            m = (rel >= 0) & (rel < _CPT)
            relc = jnp.where(m, rel, 0)
            plsc.addupdate_scatter(hist, [relc * 16 + lane], one16, mask=m)
            for u in range(2):
                i = j * 2 + u
                qidx = qsel1 + 8 * i
                a8 = plsc.load_gather(awin, [qidx])
                os = [plsc.load_gather(ac, [fl2, a8]) for ac in accs]
                vs = [plsc.load_gather(embw, [fl2 + 2 * c, qidx])
                      for c in range(4)]
                for s in range(1, 8):
                    ap = _take16(a8, lane ^ (2 * s))
                    eq = a8 == ap
                    vs = [jnp.where(eq,
                                    jnp.maximum(v, _take16(v, lane ^ (2 * s))),
                                    v)
                          for v in vs]
                for ac, o, v in zip(accs, os, vs):
                    plsc.store_scatter(ac, [fl2, a8], jnp.maximum(o, v))
            return _

        lax.fori_loop(0, _WQA // 16, oct_grp, None)

    for b in range(_B):
        def acc_init(i, _):
            for ac in accs:
                for f in range(2):
                    ac[f, pl.ds(i * 16, 16)] = neg16
            return _

        lax.fori_loop(0, _NP // 16, acc_init, None)

        def zero_hist(r, _):
            hist[pl.ds(r * 16, 16)] = zero16
            return _

        lax.fori_loop(0, _CPT, zero_hist, None)

        start(b, 0, 0)

        def wpair(wp, _):
            w0 = 2 * wp
            start(b, w0 + 1, 1)
            wait(b, w0, 0)
            process(0)

            @pl.when(wp + 1 < half)
            def _():
                start(b, w0 + 2, 0)

            wait(b, w0 + 1, 1)
            process(1)
            return _

        lax.fori_loop(0, half, wpair, None)
        for c, ac in enumerate(accs):
            pltpu.sync_copy(ac, segT.at[b].at[pl.ds(f0 + 2 * c, 2), :])
        _fold_rows(hist, cl, _CPT, 0.0)
        pltpu.sync_copy(cl, counts_out.at[b].at[pl.ds(base, _CPT)])


_stage_a_call = pl.kernel(
    _stage_a_body,
    out_type=(
        jax.ShapeDtypeStruct((_B, _D, _NP), jnp.float32),
        jax.ShapeDtypeStruct((_B, _NP), jnp.float32),
    ),
    mesh=_MESH,
    compiler_params=_SC_PARAMS,
    scratch_types=(
        pltpu.VMEM((2, _NP), jnp.float32),
        pltpu.VMEM((2, _NP), jnp.float32),
        pltpu.VMEM((2, _NP), jnp.float32),
        pltpu.VMEM((2, _NP), jnp.float32),
        pltpu.VMEM((_CPT * 16,), jnp.float32),
        pltpu.VMEM((_CPT,), jnp.float32),
        (pltpu.VMEM((_WQA,), jnp.int32), pltpu.VMEM((_WQA,), jnp.int32)),
        (pltpu.VMEM((8, _WQA), jnp.float32),
         pltpu.VMEM((8, _WQA), jnp.float32)),
        (pltpu.SemaphoreType.DMA, pltpu.SemaphoreType.DMA),
    ),
)


def _prop_body(*refs):
    g_refs = refs[0:8]
    srcs = refs[8]
    dsts = refs[9]
    ws = refs[10]
    s_refs = refs[11:19]
    (acc, rows, srcwin, dstwin, wwin, zbuf,
     isems, gsems, ssems, zsem) = refs[19:29]

    cid = lax.axis_index("c")
    sid = lax.axis_index("s")
    e0 = sid * _EPT
    zero16 = jnp.zeros((16,), jnp.float32)

    pltpu.sync_copy(ws.at[pl.ds(e0, _EPT)], wwin)

    def zero_zbuf(i, _):
        for f in range(8):
            zbuf[i, pl.ds(f * 16, 16)] = zero16
        return _

    lax.fori_loop(0, 16, zero_zbuf, None)

    def istage(w, i):
        we0 = e0 + w * _KE
        pltpu.async_copy(srcs.at[pl.ds(we0, _KE)], srcwin[i], isems[i])
        pltpu.async_copy(dsts.at[pl.ds(we0, _KE)], dstwin[i], isems[i])

    def iwait(w, i):
        we0 = e0 + w * _KE
        pltpu.make_async_copy(srcs.at[pl.ds(we0, _KE)], srcwin[i],
                              isems[i]).wait()
        pltpu.make_async_copy(dsts.at[pl.ds(we0, _KE)], dstwin[i],
                              isems[i]).wait()

    def gstart(gr, i, k):
        pltpu.async_copy(gr.at[srcwin[i]], rows[k], gsems[k])

    def gwait(gr, i, k):
        pltpu.make_async_copy(gr.at[srcwin[i]], rows[k], gsems[k]).wait()

    def scale(w, k):
        def grp(j, _):
            w16 = wwin[pl.ds(w * _KE + j * 16, 16)]
            for u in range(16):
                e = j * 16 + u
                wb = _take16(w16, jnp.full((16,), u, jnp.int32))
                for f in range(8):
                    rows[k][e, pl.ds(f * 16, 16)] = (
                        rows[k][e, pl.ds(f * 16, 16)] * wb)
            return _

        lax.fori_loop(0, _KE // 16, grp, None)

    def scat(i, k):
        pltpu.async_copy(rows[k], acc.at[dstwin[i]], ssems[k], add=True)

    def swait(i, k):
        pltpu.make_async_copy(rows[k], acc.at[dstwin[i]], ssems[k]).wait()

    for chunk in range(8):
        @pl.when(cid == chunk // 4)
        def _():
            gr = g_refs[chunk]
            sr = s_refs[chunk]

            def zero_acc(i, _):
                pltpu.async_copy(
                    zbuf, acc.at[pl.ds(sid * _ROWS_PT + i * 16, 16), :], zsem)
                return _

            lax.fori_loop(0, _ROWS_PT // 16, zero_acc, None)

            def zero_drain(i, _):
                pltpu.make_async_copy(
                    zbuf, acc.at[pl.ds(sid * _ROWS_PT + i * 16, 16), :],
                    zsem).wait()
                return _

            lax.fori_loop(0, _ROWS_PT // 16, zero_drain, None)
            plsc.subcore_barrier()

            istage(0, 0)
            iwait(0, 0)
            gstart(gr, 0, 0)
            istage(1, 1)

            def w4(wt, _):
                w0 = 4 * wt
                for u in range(4):
                    w = w0 + u
                    k = u % 2
                    iu = u
                    iun = (u + 1) % 4
                    iup = (u + 3) % 4

                    @pl.when(w + 1 < _NWIN)
                    def _():
                        iwait(w + 1, iun)

                    @pl.when(w + 2 < _NWIN)
                    def _():
                        istage(w + 2, (u + 2) % 4)

                    gwait(gr, iu, k)

                    @pl.when(w >= 1)
                    def _():
                        swait(iup, 1 - k)

                    @pl.when(w + 1 < _NWIN)
                    def _():
                        gstart(gr, iun, 1 - k)

                    scale(w, k)
                    scat(iu, k)
                return _

            lax.fori_loop(0, _NWIN // 4, w4, None)
            swait(3, 1)
            plsc.subcore_barrier()
            pltpu.sync_copy(acc.at[pl.ds(sid * _ROWS_PT, _ROWS_PT), :],
                            sr.at[pl.ds(sid * _ROWS_PT, _ROWS_PT), :])
            plsc.subcore_barrier()


_prop_call = pl.kernel(
    _prop_body,
    out_type=tuple(jax.ShapeDtypeStruct((_NP, 128), jnp.float32)
                   for _ in range(8)),
    mesh=_MESH,
    compiler_params=_SC_PARAMS,
    scratch_types=(
        pltpu.VMEM_SHARED((_ACC_ROWS, 128), jnp.float32),
        (pltpu.VMEM((_KE, 128), jnp.float32),
         pltpu.VMEM((_KE, 128), jnp.float32)),
        tuple(pltpu.VMEM((_KE,), jnp.int32) for _ in range(4)),
        tuple(pltpu.VMEM((_KE,), jnp.int32) for _ in range(4)),
        pltpu.VMEM((_EPT,), jnp.float32),
        pltpu.VMEM((16, 128), jnp.float32),
        tuple(pltpu.SemaphoreType.DMA for _ in range(4)),
        (pltpu.SemaphoreType.DMA, pltpu.SemaphoreType.DMA),
        (pltpu.SemaphoreType.DMA, pltpu.SemaphoreType.DMA),
        pltpu.SemaphoreType.DMA,
    ),
)


_BM = 512
_GRID = _NP // _BM


def _mm1_body(seg_ref, cnt_ref, dummy_ref, dinv_ref, w_ref, *out_refs):
    w = w_ref[...]
    dummy = dummy_ref[...]
    dinv = dinv_ref[...]
    for b in range(_B):
        x = seg_ref[b]
        cnt = cnt_ref[pl.ds(b, 1), :]
        xm = jnp.where(cnt < float(_CAP), jnp.maximum(x, dummy), x)
        h = lax.dot_general(xm, w, (((0,), (0,)), ((), ())),
                            preferred_element_type=jnp.float32)
        g = h * dinv
        out_refs[2 * b][...] = g[:, :128]
        out_refs[2 * b + 1][...] = g[:, 128:]


def _mm1(segT, counts, dummy2d, dinv2d, W1):
    return pl.pallas_call(
        _mm1_body,
        grid=(_GRID,),
        in_specs=[
            pl.BlockSpec((_B, _D, _BM), lambda m: (0, 0, m)),
            pl.BlockSpec((_B, _BM), lambda m: (0, m)),
            pl.BlockSpec((_D, 1), lambda m: (0, 0)),
            pl.BlockSpec((_BM, 1), lambda m: (m, 0)),
            pl.BlockSpec((_D, _D), lambda m: (0, 0)),
        ],
        out_specs=[pl.BlockSpec((_BM, 128), lambda m: (m, 0))
                   for _ in range(8)],
        out_shape=[jax.ShapeDtypeStruct((_NP, 128), jnp.float32)
                   for _ in range(8)],
    )(segT, counts, dummy2d, dinv2d, W1)


def _mm2_body(dinv_ref, bias_ref, w_ref, *refs):
    s_refs = refs[0:8]
    g_refs = refs[8:16]
    out_refs = refs[16:24]
    w = w_ref[...]
    bias = bias_ref[...]
    dinv = dinv_ref[...]
    for b in range(_B):
        s = jnp.concatenate([s_refs[2 * b][...], s_refs[2 * b + 1][...]],
                            axis=1)
        g = jnp.concatenate([g_refs[2 * b][...], g_refs[2 * b + 1][...]],
                            axis=1)
        x = jnp.maximum(dinv * (s + g) + bias, 0.0)
        h = lax.dot_general(x, w, (((1,), (0,)), ((), ())),
                            preferred_element_type=jnp.float32)
        g2 = h * dinv
        out_refs[2 * b][...] = g2[:, :128]
        out_refs[2 * b + 1][...] = g2[:, 128:]


def _mm2(s_list, g_list, dinv2d, bias2d, W2):
    blk = pl.BlockSpec((_BM, 128), lambda m: (m, 0))
    return pl.pallas_call(
        _mm2_body,
        grid=(_GRID,),
        in_specs=[
            pl.BlockSpec((_BM, 1), lambda m: (m, 0)),
            pl.BlockSpec((1, _D), lambda m: (0, 0)),
            pl.BlockSpec((_D, _D), lambda m: (0, 0)),
        ] + [blk] * 16,
        out_specs=[pl.BlockSpec((_BM, 128), lambda m: (m, 0))
                   for _ in range(8)],
        out_shape=[jax.ShapeDtypeStruct((_NP, 128), jnp.float32)
                   for _ in range(8)],
    )(dinv2d, bias2d, W2, *s_list, *g_list)


def _fin_body(dinv_ref, bias_ref, *refs):
    s_refs = refs[0:8]
    g_refs = refs[8:16]
    out_ref = refs[16]
    bias = bias_ref[...]
    dinv = dinv_ref[...]
    for b in range(_B):
        s = jnp.concatenate([s_refs[2 * b][...], s_refs[2 * b + 1][...]],
                            axis=1)
        g = jnp.concatenate([g_refs[2 * b][...], g_refs[2 * b + 1][...]],
                            axis=1)
        out_ref[b, :, :] = jnp.maximum(dinv * (s + g) + bias, 0.0)


def _fin(s_list, g_list, dinv2d, bias2d):
    blk = pl.BlockSpec((_BM, 128), lambda m: (m, 0))
    return pl.pallas_call(
        _fin_body,
        grid=(_GRID,),
        in_specs=[
            pl.BlockSpec((_BM, 1), lambda m: (m, 0)),
            pl.BlockSpec((1, _D), lambda m: (0, 0)),
        ] + [blk] * 16,
        out_specs=pl.BlockSpec((_B, _BM, _D), lambda m: (0, m, 0)),
        out_shape=jax.ShapeDtypeStruct((_B, _NP, _D), jnp.float32),
    )(dinv2d, bias2d, *s_list, *g_list)


def kernel(core_allocs, qubit_embs, dummy_qubit_emb, edge_index, edge_weight,
           W1, b1, W2, b2):
    src = edge_index[0].astype(jnp.int32)
    dst = edge_index[1].astype(jnp.int32)
    qpad = _QP - _Q
    alloc_pad = _N + (jnp.arange(qpad, dtype=jnp.int32) % (_NP - _N))
    allocs = jnp.concatenate(
        [core_allocs.astype(jnp.int32),
         jnp.broadcast_to(alloc_pad, (_B, qpad))], axis=1)
    embT = jnp.concatenate(
        [qubit_embs, jnp.zeros((qpad, _D), jnp.float32)], axis=0).T

    deg = _stats_call(dst, edge_weight)
    dinv2d = lax.rsqrt(deg)[:, None]

    segT, counts = _stage_a_call(allocs, embT)

    g_list = _mm1(segT, counts, dummy_qubit_emb[:, None], dinv2d, W1)

    npad = _E2 - _E
    pad_idx = (jnp.arange(npad, dtype=jnp.int32) * 16) % _N
    src2 = jnp.concatenate([src, pad_idx])
    dst2 = jnp.concatenate([dst, pad_idx])
    w2 = jnp.concatenate([edge_weight, jnp.zeros((npad,), jnp.float32)])

    s_list = _prop_call(*g_list, src2, dst2, w2)
    g2_list = _mm2(s_list, g_list, dinv2d, b1[None, :], W2)
    s2_list = _prop_call(*g2_list, src2, dst2, w2)
    return _fin(s2_list, g2_list, dinv2d, b2[None, :])[:, :_N, :]


# speedup vs baseline: 1.2509x; 1.0235x over previous
import functools
import numpy as np

import jax
import jax.numpy as jnp
from jax import lax
from jax.experimental import pallas as pl
from jax.experimental.pallas import tpu as pltpu
from jax.experimental.pallas import tpu_sc as plsc

_N = 10000
_NP = 12288
_Q = 100000
_QP = 102400
_D = 256
_B = 4
_E = 160000
_CAP = 32

_NC = 2
_NS = 16
_NW = _NC * _NS

_CPT = _NP // _NW
_WQ = 2048
_WQA = 1024
_WE = 3200

_KE = 128
_NWIN = 80
_EPT = _KE * _NWIN
_E2 = _EPT * _NS
_ACC_ROWS = 10240
_ROWS_PT = _ACC_ROWS // _NS

_LANE = lambda: lax.broadcasted_iota(jnp.int32, (16,), 0)

_GDN = lax.GatherDimensionNumbers(offset_dims=(), collapsed_slice_dims=(0,),
                                  start_index_map=(0,))


def _take16(x, idx):
    return lax.gather(x, idx[:, None], _GDN, (1,),
                      mode=lax.GatherScatterMode.PROMISE_IN_BOUNDS)

_MESH = plsc.VectorSubcoreMesh(core_axis_name="c", subcore_axis_name="s")
_SC_PARAMS = pltpu.CompilerParams(needs_layout_passes=False)

_NEGINF = float("-inf")


def _wid():
    return lax.axis_index("s") * _NC + lax.axis_index("c")


def _fold_rows(hist_ref, out_ref, n_rows, extra):
    lane = _LANE()
    m0 = lane == 0
    perms = [lane ^ s for s in (1, 2, 4, 8)]

    def fold_body(r, _):
        x = hist_ref[pl.ds(r * 16, 16)]
        for p in perms:
            x = x + _take16(x, p)
        plsc.store_scatter(out_ref, [jnp.broadcast_to(r, (16,))], x + extra,
                           mask=m0)
        return _

    lax.fori_loop(0, n_rows, fold_body, None)


def _stats_body(dsts, ws, deg_out, hist, dwin, wwin, cl, sems):
    wid = _wid()
    base = wid * _CPT
    lane = _LANE()
    zero16 = jnp.zeros((16,), jnp.float32)
    half = _E // _WE // 2

    def zero_hist(r, _):
        hist[pl.ds(r * 16, 16)] = zero16
        return _

    lax.fori_loop(0, _CPT, zero_hist, None)

    def start(w, k):
        e0 = w * _WE
        pltpu.async_copy(dsts.at[pl.ds(e0, _WE)], dwin[k], sems[k])
        pltpu.async_copy(ws.at[pl.ds(e0, _WE)], wwin[k], sems[k])

    def wait(w, k):
        e0 = w * _WE
        pltpu.make_async_copy(dsts.at[pl.ds(e0, _WE)], dwin[k],
                              sems[k]).wait()
        pltpu.make_async_copy(ws.at[pl.ds(e0, _WE)], wwin[k],
                              sems[k]).wait()

    def process(k):
        def grp(j, _):
            dv = dwin[k][pl.ds(j * 16, 16)]
            wv = wwin[k][pl.ds(j * 16, 16)]
            rel = dv - base
            m = (rel >= 0) & (rel < _CPT)
            relc = jnp.where(m, rel, 0)
            plsc.addupdate_scatter(hist, [relc * 16 + lane], wv, mask=m)
            return _

        lax.fori_loop(0, _WE // 16, grp, None)

    start(0, 0)

    def wpair(wp, _):
        w0 = 2 * wp
        start(w0 + 1, 1)
        wait(w0, 0)
        process(0)

        @pl.when(wp + 1 < half)
        def _():
            start(w0 + 2, 0)

        wait(w0 + 1, 1)
        process(1)
        return _

    lax.fori_loop(0, half, wpair, None)
    _fold_rows(hist, cl, _CPT, 1.0)
    pltpu.sync_copy(cl, deg_out.at[pl.ds(base, _CPT)])


_stats_call = pl.kernel(
    _stats_body,
    out_type=jax.ShapeDtypeStruct((_NP,), jnp.float32),
    mesh=_MESH,
    compiler_params=_SC_PARAMS,
    scratch_types=(
        pltpu.VMEM((_CPT * 16,), jnp.float32),
        (pltpu.VMEM((_WE,), jnp.int32), pltpu.VMEM((_WE,), jnp.int32)),
        (pltpu.VMEM((_WE,), jnp.float32), pltpu.VMEM((_WE,), jnp.float32)),
        pltpu.VMEM((_CPT,), jnp.float32),
        (pltpu.SemaphoreType.DMA, pltpu.SemaphoreType.DMA),
    ),
)


def _stage_a_body(allocs, embT, segT, counts_out, acc0, acc1, acc2, acc3,
                  hist, cl, aw, ew, sems):
    wid = _wid()
    f0 = wid * 8
    base = wid * _CPT
    neg16 = jnp.full((16,), _NEGINF, jnp.float32)
    zero16 = jnp.zeros((16,), jnp.float32)
    one16 = jnp.full((16,), 1.0, jnp.float32)
    lane = _LANE()
    qsel1 = lax.shift_right_logical(lane, 1)
    fl2 = lane & 1
    accs = (acc0, acc1, acc2, acc3)
    nwin = _QP // _WQA
    half = nwin // 2

    def start(b, w, k):
        q0 = w * _WQA
        pltpu.async_copy(allocs.at[b].at[pl.ds(q0, _WQA)], aw[k], sems[k])
        pltpu.async_copy(embT.at[pl.ds(f0, 8), pl.ds(q0, _WQA)], ew[k],
                         sems[k])

    def wait(b, w, k):
        q0 = w * _WQA
        pltpu.make_async_copy(allocs.at[b].at[pl.ds(q0, _WQA)], aw[k],
                              sems[k]).wait()
        pltpu.make_async_copy(embT.at[pl.ds(f0, 8), pl.ds(q0, _WQA)], ew[k],
                              sems[k]).wait()

    def process(k):
        awin = aw[k]
        embw = ew[k]

        def oct_grp(j, _):
            av = awin[pl.ds(j * 16, 16)]
            rel = av - base
            m = (rel >= 0) & (rel < _CPT)
            relc = jnp.where(m, rel, 0)
            plsc.addupdate_scatter(hist, [relc * 16 + lane], one16, mask=m)
            for u in range(2):
                i = j * 2 + u
                qidx = qsel1 + 8 * i
                a8 = plsc.load_gather(awin, [qidx])
                os = [plsc.load_gather(ac, [fl2, a8]) for ac in accs]
                vs = [plsc.load_gather(embw, [fl2 + 2 * c, qidx])
                      for c in range(4)]
                for s in range(1, 8):
                    ap = _take16(a8, lane ^ (2 * s))
                    eq = a8 == ap
                    vs = [jnp.where(eq,
                                    jnp.maximum(v, _take16(v, lane ^ (2 * s))),
                                    v)
                          for v in vs]
                for ac, o, v in zip(accs, os, vs):
                    plsc.store_scatter(ac, [fl2, a8], jnp.maximum(o, v))
            return _

        lax.fori_loop(0, _WQA // 16, oct_grp, None)

    for b in range(_B):
        def acc_init(i, _):
            for ac in accs:
                for f in range(2):
                    ac[f, pl.ds(i * 16, 16)] = neg16
            return _

        lax.fori_loop(0, _NP // 16, acc_init, None)

        def zero_hist(r, _):
            hist[pl.ds(r * 16, 16)] = zero16
            return _

        lax.fori_loop(0, _CPT, zero_hist, None)

        start(b, 0, 0)

        def wpair(wp, _):
            w0 = 2 * wp
            start(b, w0 + 1, 1)
            wait(b, w0, 0)
            process(0)

            @pl.when(wp + 1 < half)
            def _():
                start(b, w0 + 2, 0)

            wait(b, w0 + 1, 1)
            process(1)
            return _

        lax.fori_loop(0, half, wpair, None)
        for c, ac in enumerate(accs):
            pltpu.sync_copy(ac, segT.at[b].at[pl.ds(f0 + 2 * c, 2), :])
        _fold_rows(hist, cl, _CPT, 0.0)
        pltpu.sync_copy(cl, counts_out.at[b].at[pl.ds(base, _CPT)])


_stage_a_call = pl.kernel(
    _stage_a_body,
    out_type=(
        jax.ShapeDtypeStruct((_B, _D, _NP), jnp.float32),
        jax.ShapeDtypeStruct((_B, _NP), jnp.float32),
    ),
    mesh=_MESH,
    compiler_params=_SC_PARAMS,
    scratch_types=(
        pltpu.VMEM((2, _NP), jnp.float32),
        pltpu.VMEM((2, _NP), jnp.float32),
        pltpu.VMEM((2, _NP), jnp.float32),
        pltpu.VMEM((2, _NP), jnp.float32),
        pltpu.VMEM((_CPT * 16,), jnp.float32),
        pltpu.VMEM((_CPT,), jnp.float32),
        (pltpu.VMEM((_WQA,), jnp.int32), pltpu.VMEM((_WQA,), jnp.int32)),
        (pltpu.VMEM((8, _WQA), jnp.float32),
         pltpu.VMEM((8, _WQA), jnp.float32)),
        (pltpu.SemaphoreType.DMA, pltpu.SemaphoreType.DMA),
    ),
)


def _prop_body(*refs):
    g_refs = refs[0:8]
    srcs = refs[8]
    dsts = refs[9]
    ws = refs[10]
    s_refs = refs[11:19]
    (acc, rows, srcwin, dstwin, wwin, zbuf,
     isems, gsems, ssems, zsem) = refs[19:29]

    cid = lax.axis_index("c")
    sid = lax.axis_index("s")
    e0 = sid * _EPT
    zero16 = jnp.zeros((16,), jnp.float32)

    pltpu.sync_copy(ws.at[pl.ds(e0, _EPT)], wwin)

    def zero_zbuf(i, _):
        for f in range(8):
            zbuf[i, pl.ds(f * 16, 16)] = zero16
        return _

    lax.fori_loop(0, 16, zero_zbuf, None)

    def istage(w, i):
        we0 = e0 + w * _KE
        pltpu.async_copy(srcs.at[pl.ds(we0, _KE)], srcwin[i], isems[i])
        pltpu.async_copy(dsts.at[pl.ds(we0, _KE)], dstwin[i], isems[i])

    def iwait(w, i):
        we0 = e0 + w * _KE
        pltpu.make_async_copy(srcs.at[pl.ds(we0, _KE)], srcwin[i],
                              isems[i]).wait()
        pltpu.make_async_copy(dsts.at[pl.ds(we0, _KE)], dstwin[i],
                              isems[i]).wait()

    def gstart(gr, i, k):
        pltpu.async_copy(gr.at[srcwin[i]], rows[k], gsems[k])

    def gwait(gr, i, k):
        pltpu.make_async_copy(gr.at[srcwin[i]], rows[k], gsems[k]).wait()

    def scale(w, k):
        def grp(j, _):
            w16 = wwin[pl.ds(w * _KE + j * 16, 16)]
            for u in range(16):
                e = j * 16 + u
                wb = _take16(w16, jnp.full((16,), u, jnp.int32))
                for f in range(8):
                    rows[k][e, pl.ds(f * 16, 16)] = (
                        rows[k][e, pl.ds(f * 16, 16)] * wb)
            return _

        lax.fori_loop(0, _KE // 16, grp, None)

    def scat(i, k):
        pltpu.async_copy(rows[k], acc.at[dstwin[i]], ssems[k], add=True)

    def swait(i, k):
        pltpu.make_async_copy(rows[k], acc.at[dstwin[i]], ssems[k]).wait()

    for chunk in range(8):
        @pl.when(cid == chunk // 4)
        def _():
            gr = g_refs[chunk]
            sr = s_refs[chunk]

            def zero_acc(i, _):
                pltpu.async_copy(
                    zbuf, acc.at[pl.ds(sid * _ROWS_PT + i * 16, 16), :], zsem)
                return _

            lax.fori_loop(0, _ROWS_PT // 16, zero_acc, None)

            def zero_drain(i, _):
                pltpu.make_async_copy(
                    zbuf, acc.at[pl.ds(sid * _ROWS_PT + i * 16, 16), :],
                    zsem).wait()
                return _

            lax.fori_loop(0, _ROWS_PT // 16, zero_drain, None)
            plsc.subcore_barrier()

            istage(0, 0)
            iwait(0, 0)
            gstart(gr, 0, 0)
            istage(1, 1)

            def w4(wt, _):
                w0 = 4 * wt
                for u in range(4):
                    w = w0 + u
                    k = u % 2
                    iu = u
                    iun = (u + 1) % 4
                    iup = (u + 3) % 4

                    @pl.when(w + 1 < _NWIN)
                    def _():
                        iwait(w + 1, iun)

                    @pl.when(w + 2 < _NWIN)
                    def _():
                        istage(w + 2, (u + 2) % 4)

                    gwait(gr, iu, k)

                    @pl.when(w >= 1)
                    def _():
                        swait(iup, 1 - k)

                    @pl.when(w + 1 < _NWIN)
                    def _():
                        gstart(gr, iun, 1 - k)

                    scale(w, k)
                    scat(iu, k)
                return _

            lax.fori_loop(0, _NWIN // 4, w4, None)
            swait(3, 1)
            plsc.subcore_barrier()
            pltpu.sync_copy(acc.at[pl.ds(sid * _ROWS_PT, _ROWS_PT), :],
                            sr.at[pl.ds(sid * _ROWS_PT, _ROWS_PT), :])
            plsc.subcore_barrier()


_prop_call = pl.kernel(
    _prop_body,
    out_type=tuple(jax.ShapeDtypeStruct((_NP, 128), jnp.float32)
                   for _ in range(8)),
    mesh=_MESH,
    compiler_params=_SC_PARAMS,
    scratch_types=(
        pltpu.VMEM_SHARED((_ACC_ROWS, 128), jnp.float32),
        (pltpu.VMEM((_KE, 128), jnp.float32),
         pltpu.VMEM((_KE, 128), jnp.float32)),
        tuple(pltpu.VMEM((_KE,), jnp.int32) for _ in range(4)),
        tuple(pltpu.VMEM((_KE,), jnp.int32) for _ in range(4)),
        pltpu.VMEM((_EPT,), jnp.float32),
        pltpu.VMEM((16, 128), jnp.float32),
        tuple(pltpu.SemaphoreType.DMA for _ in range(4)),
        (pltpu.SemaphoreType.DMA, pltpu.SemaphoreType.DMA),
        (pltpu.SemaphoreType.DMA, pltpu.SemaphoreType.DMA),
        pltpu.SemaphoreType.DMA,
    ),
)


_BM = 512
_GRID = _NP // _BM


def _mm1_body(seg_ref, cnt_ref, dummy_ref, dinv_ref, w_ref, *out_refs):
    w = w_ref[...]
    dummy = dummy_ref[...]
    dinv = dinv_ref[...]
    for b in range(_B):
        x = seg_ref[b]
        cnt = cnt_ref[pl.ds(b, 1), :]
        xm = jnp.where(cnt < float(_CAP), jnp.maximum(x, dummy), x)
        h = lax.dot_general(xm, w, (((0,), (0,)), ((), ())),
                            preferred_element_type=jnp.float32)
        g = h * dinv
        out_refs[2 * b][...] = g[:, :128]
        out_refs[2 * b + 1][...] = g[:, 128:]


def _mm1(segT, counts, dummy2d, dinv2d, W1):
    return pl.pallas_call(
        _mm1_body,
        grid=(_GRID,),
        in_specs=[
            pl.BlockSpec((_B, _D, _BM), lambda m: (0, 0, m)),
            pl.BlockSpec((_B, _BM), lambda m: (0, m)),
            pl.BlockSpec((_D, 1), lambda m: (0, 0)),
            pl.BlockSpec((_BM, 1), lambda m: (m, 0)),
            pl.BlockSpec((_D, _D), lambda m: (0, 0)),
        ],
        out_specs=[pl.BlockSpec((_BM, 128), lambda m: (m, 0))
                   for _ in range(8)],
        out_shape=[jax.ShapeDtypeStruct((_NP, 128), jnp.float32)
                   for _ in range(8)],
    )(segT, counts, dummy2d, dinv2d, W1)


def _mm2_body(dinv_ref, bias_ref, w_ref, *refs):
    s_refs = refs[0:8]
    g_refs = refs[8:16]
    out_refs = refs[16:24]
    w = w_ref[...]
    bias = bias_ref[...]
    dinv = dinv_ref[...]
    for b in range(_B):
        s = jnp.concatenate([s_refs[2 * b][...], s_refs[2 * b + 1][...]],
                            axis=1)
        g = jnp.concatenate([g_refs[2 * b][...], g_refs[2 * b + 1][...]],
                            axis=1)
        x = jnp.maximum(dinv * (s + g) + bias, 0.0)
        h = lax.dot_general(x, w, (((1,), (0,)), ((), ())),
                            preferred_element_type=jnp.float32)
        g2 = h * dinv
        out_refs[2 * b][...] = g2[:, :128]
        out_refs[2 * b + 1][...] = g2[:, 128:]


def _mm2(s_list, g_list, dinv2d, bias2d, W2):
    blk = pl.BlockSpec((_BM, 128), lambda m: (m, 0))
    return pl.pallas_call(
        _mm2_body,
        grid=(_GRID,),
        in_specs=[
            pl.BlockSpec((_BM, 1), lambda m: (m, 0)),
            pl.BlockSpec((1, _D), lambda m: (0, 0)),
            pl.BlockSpec((_D, _D), lambda m: (0, 0)),
        ] + [blk] * 16,
        out_specs=[pl.BlockSpec((_BM, 128), lambda m: (m, 0))
                   for _ in range(8)],
        out_shape=[jax.ShapeDtypeStruct((_NP, 128), jnp.float32)
                   for _ in range(8)],
    )(dinv2d, bias2d, W2, *s_list, *g_list)


def _fin_body(dinv_ref, bias_ref, *refs):
    s_refs = refs[0:8]
    g_refs = refs[8:16]
    out_ref = refs[16]
    bias = bias_ref[...]
    dinv = dinv_ref[...]
    for b in range(_B):
        s = jnp.concatenate([s_refs[2 * b][...], s_refs[2 * b + 1][...]],
                            axis=1)
        g = jnp.concatenate([g_refs[2 * b][...], g_refs[2 * b + 1][...]],
                            axis=1)
        out_ref[b, :, :] = jnp.maximum(dinv * (s + g) + bias, 0.0)


def _fin(s_list, g_list, dinv2d, bias2d):
    blk = pl.BlockSpec((_BM, 128), lambda m: (m, 0))
    return pl.pallas_call(
        _fin_body,
        grid=(_GRID,),
        in_specs=[
            pl.BlockSpec((_BM, 1), lambda m: (m, 0)),
            pl.BlockSpec((1, _D), lambda m: (0, 0)),
        ] + [blk] * 16,
        out_specs=pl.BlockSpec((_B, _BM, _D), lambda m: (0, m, 0)),
        out_shape=jax.ShapeDtypeStruct((_B, _NP, _D), jnp.float32),
    )(dinv2d, bias2d, *s_list, *g_list)


def kernel(core_allocs, qubit_embs, dummy_qubit_emb, edge_index, edge_weight,
           W1, b1, W2, b2):
    src = edge_index[0].astype(jnp.int32)
    dst = edge_index[1].astype(jnp.int32)
    qpad = _QP - _Q
    alloc_pad = _N + (jnp.arange(qpad, dtype=jnp.int32) % (_NP - _N))
    allocs = jnp.concatenate(
        [core_allocs.astype(jnp.int32),
         jnp.broadcast_to(alloc_pad, (_B, qpad))], axis=1)
    embT = jnp.concatenate(
        [qubit_embs, jnp.zeros((qpad, _D), jnp.float32)], axis=0).T

    deg = _stats_call(dst, edge_weight)
    dinv2d = lax.rsqrt(deg)[:, None]

    segT, counts = _stage_a_call(allocs, embT)

    g_list = _mm1(segT, counts, dummy_qubit_emb[:, None], dinv2d, W1)

    npad = _E2 - _E
    pad_idx = (jnp.arange(npad, dtype=jnp.int32) * 16) % _N
    src2 = jnp.concatenate([src, pad_idx])
    dst2 = jnp.concatenate([dst, pad_idx])
    w2 = jnp.concatenate([edge_weight, jnp.zeros((npad,), jnp.float32)])

    s_list = _prop_call(*g_list, src2, dst2, w2)
    g2_list = _mm2(s_list, g_list, dinv2d, b1[None, :], W2)
    s2_list = _prop_call(*g2_list, src2, dst2, w2)
    return _fin(s2_list, g2_list, dinv2d, b2[None, :])[:, :_N, :]
